# Initial kernel scaffold; baseline (speedup 1.0000x reference)
import jax
import jax.numpy as jnp
from jax.experimental import pallas as pl


def kernel(edges, alpha, beta, kappa, snakes, target_mask, target_snake):
    raise NotImplementedError("write your pallas kernel here")


# speedup vs baseline: 2.2394x; 2.2394x over previous
import jax
import jax.numpy as jnp
from jax.experimental import pallas as pl
from jax.experimental.pallas import tpu as pltpu

_B, _M, _N, _L = 8, 256, 256, 128
_ECHUNK = 8


def _roll_m1(a):
    return jnp.concatenate([a[:, 1:], a[:, :1]], axis=1)


def _roll_p1(a):
    return jnp.concatenate([a[:, -1:], a[:, :-1]], axis=1)


def _mean_der1(u, v):
    d1u = _roll_m1(u) - _roll_p1(u)
    d1v = _roll_m1(v) - _roll_p1(v)
    return jnp.mean(jnp.sqrt(d1u * d1u + d1v * d1v))


def _rasterize(u, v):
    r, c = u, v
    r2, c2 = _roll_m1(u), _roll_m1(v)
    yy = jax.lax.broadcasted_iota(jnp.int32, (_M, 1), 0).astype(jnp.float32)
    cond = (r > yy) != (r2 > yy)
    denom = jnp.where(jnp.abs(r2 - r) < 1e-9, 1e-9, r2 - r)
    xint = c + (c2 - c) * (yy - r) / denom
    xint = jnp.where(cond, xint, -1.0)
    xx = jax.lax.broadcasted_iota(
        jnp.int32, (_M, _ECHUNK, _N), 2).astype(jnp.float32)
    count = jnp.zeros((_M, _N), jnp.int32)
    for k in range(_L // _ECHUNK):
        xk = xint[:, k * _ECHUNK:(k + 1) * _ECHUNK]
        cross = xx < xk[:, :, None]
        count = count + jnp.sum(cross.astype(jnp.int32), axis=1)
    return (count & 1).astype(jnp.float32)


def _loss_kernel(su_v, sv_v, tu_v, tv_v, su_s, sv_s, tu_s, tv_s, tgt_ref,
                 ge_ref, ga_ref, gb_ref, gk_ref,
                 iou_ref, inter_ref, uni_ref, agt_ref, asn_ref,
                 t1_scr, t2_scr):
    su = su_v[0]
    sv = sv_v[0]
    tu = tu_v[0]
    tv = tv_v[0]
    tgt = tgt_ref[0]

    mask = _rasterize(su, sv)
    s = tgt + mask
    isum = jnp.sum((s == 2.0).astype(jnp.int32))
    usum = jnp.sum((s >= 1.0).astype(jnp.int32))
    isum_f = isum.astype(jnp.float32)
    usum_f = usum.astype(jnp.float32)
    iou_ref[0, 0, 0] = isum_f / jnp.maximum(usum_f, 1.0)
    inter_ref[0, 0, 0] = isum_f / float(_M * _N)
    uni_ref[0, 0, 0] = usum_f / float(_M * _N)
    agt_ref[0, 0, 0] = jnp.sum((tgt > 0).astype(jnp.int32))
    asn_ref[0, 0, 0] = jnp.sum((mask > 0).astype(jnp.int32))

    gk_ref[0] = tgt - mask

    const = _mean_der1(su, sv) - _mean_der1(tu, tv)
    ga_ref[0] = jnp.zeros((_M, _N), jnp.float32) + const

    ge_ref[0] = jnp.zeros((_M, _N), jnp.float32)
    gb_ref[0] = jnp.zeros((_M, _N), jnp.float32)
    t1_scr[:] = jnp.zeros((_M, _N), jnp.float32)
    t2_scr[:] = jnp.zeros((_M, _N), jnp.float32)

    _W = 16
    ly_base = jax.lax.broadcasted_iota(jnp.int32, (_W, _N), 0).astype(jnp.float32)
    xxp = jax.lax.broadcasted_iota(jnp.int32, (_W, _N), 1).astype(jnp.float32)

    def body(k, _):
        km = jnp.where(k == 0, _L - 1, k - 1)
        kp = jnp.where(k == _L - 1, 0, k + 1)

        def stamp(us_ref, vs_ref, store1, store2):
            uk = us_ref[0, 0, k]
            vk = vs_ref[0, 0, k]
            ukm = us_ref[0, 0, km]
            ukp = us_ref[0, 0, kp]
            vkm = vs_ref[0, 0, km]
            vkp = vs_ref[0, 0, kp]
            d2u = ukp + ukm - 2.0 * uk
            d2v = vkp + vkm - 2.0 * vk
            d2k = jnp.sqrt(d2u * d2u + d2v * d2v)
            blk = jnp.clip((jnp.floor(uk).astype(jnp.int32) - 2) >> 3,
                           0, (_M - _W) // 8)
            row0 = blk * 8
            ly = row0.astype(jnp.float32) + ly_base
            dy = ly - uk
            dx = xxp - vk
            dist2 = dy * dy + dx * dx
            m = dist2 <= 4.0
            p1 = jnp.where(m, 1.0, 0.0)
            p2 = jnp.where(m, d2k, 0.0)
            store1(row0, p1)
            store2(row0, p2)

        def st_ge(row0, p):
            ge_ref[0, pl.ds(row0, _W), :] = jnp.maximum(
                ge_ref[0, pl.ds(row0, _W), :], p)

        def st_gb(row0, p):
            gb_ref[0, pl.ds(row0, _W), :] = jnp.maximum(
                gb_ref[0, pl.ds(row0, _W), :], p)

        def st_t1(row0, p):
            t1_scr[pl.ds(row0, _W), :] = jnp.maximum(
                t1_scr[pl.ds(row0, _W), :], p)

        def st_t2(row0, p):
            t2_scr[pl.ds(row0, _W), :] = jnp.maximum(
                t2_scr[pl.ds(row0, _W), :], p)

        stamp(su_s, sv_s, st_ge, st_gb)
        stamp(tu_s, tv_s, st_t1, st_t2)
        return 0

    jax.lax.fori_loop(0, _L, body, 0)

    ge_ref[0] = ge_ref[0] - t1_scr[:]
    gb_ref[0] = gb_ref[0] - t2_scr[:]


def _run(su3, sv3, tu3, tv3, tgt3, interpret=False):
    poly_vmem = pl.BlockSpec((1, 1, _L), lambda i: (i, 0, 0))
    poly_smem = pl.BlockSpec((1, 1, _L), lambda i: (i, 0, 0),
                             memory_space=pltpu.SMEM)
    img_spec = pl.BlockSpec((1, _M, _N), lambda i: (i, 0, 0))
    met_spec = pl.BlockSpec((1, 1, 1), lambda i: (i, 0, 0),
                            memory_space=pltpu.SMEM)
    f32 = jnp.float32
    i32 = jnp.int32
    out_shape = (
        jax.ShapeDtypeStruct((_B, _M, _N), f32),
        jax.ShapeDtypeStruct((_B, _M, _N), f32),
        jax.ShapeDtypeStruct((_B, _M, _N), f32),
        jax.ShapeDtypeStruct((_B, _M, _N), f32),
        jax.ShapeDtypeStruct((_B, 1, 1), f32),
        jax.ShapeDtypeStruct((_B, 1, 1), f32),
        jax.ShapeDtypeStruct((_B, 1, 1), f32),
        jax.ShapeDtypeStruct((_B, 1, 1), i32),
        jax.ShapeDtypeStruct((_B, 1, 1), i32),
    )
    return pl.pallas_call(
        _loss_kernel,
        grid=(_B,),
        in_specs=[poly_vmem, poly_vmem, poly_vmem, poly_vmem,
                  poly_smem, poly_smem, poly_smem, poly_smem, img_spec],
        out_specs=(img_spec, img_spec, img_spec, img_spec,
                   met_spec, met_spec, met_spec, met_spec, met_spec),
        out_shape=out_shape,
        scratch_shapes=[pltpu.VMEM((_M, _N), f32),
                        pltpu.VMEM((_M, _N), f32)],
        interpret=interpret,
    )(su3, sv3, tu3, tv3, su3, sv3, tu3, tv3, tgt3)


def kernel(edges, alpha, beta, kappa, snakes, target_mask, target_snake,
           interpret=False):
    Bn, _, m, n = edges.shape
    su3 = snakes[:, :, 0].reshape(Bn, 1, _L)
    sv3 = snakes[:, :, 1].reshape(Bn, 1, _L)
    tu3 = target_snake[:, :, 0].reshape(Bn, 1, _L)
    tv3 = target_snake[:, :, 1].reshape(Bn, 1, _L)
    tgt3 = target_mask.reshape(Bn, m, n)
    ge, ga, gb, gk, iou, inter, uni, agt, asn = _run(
        su3, sv3, tu3, tv3, tgt3, interpret=interpret)
    return (ge[:, None], ga[:, None], gb[:, None], gk[:, None],
            iou.reshape(Bn), inter.reshape(Bn), uni.reshape(Bn),
            agt.reshape(Bn), asn.reshape(Bn))
